# Initial kernel scaffold; baseline (speedup 1.0000x reference)
import jax
import jax.numpy as jnp
from jax.experimental import pallas as pl

HIDDEN = 2048
NQ = 16
NKV = 4
DH = 128
G = NQ // NKV
KERNEL_W = 32
STRIDE = 16
WIN = 512
THETA = 500000.0
T = 2048
NUM_C = (T - KERNEL_W) // STRIDE + 1
C_PAD = 128
QT = 256


def _llama3_inv_freq():
    inv = 1.0 / (THETA ** (jnp.arange(0, DH, 2, dtype=jnp.float32) / DH))
    factor, lo, hi, orig = 8.0, 1.0, 4.0, 8192.0
    wavelen = 2.0 * jnp.pi / inv
    smooth = jnp.clip((orig / wavelen - lo) / (hi - lo), 0.0, 1.0)
    return jnp.where(
        wavelen > orig / lo,
        inv / factor,
        jnp.where(wavelen < orig / hi, inv, (1.0 - smooth) * inv / factor + smooth * inv),
    )


def _rope(x, pos):
    inv = _llama3_inv_freq()
    f = pos[:, None].astype(jnp.float32) * inv[None, :]
    cos = jnp.cos(f)[:, None, :]
    sin = jnp.sin(f)[:, None, :]
    x1 = x[..., ::2]
    x2 = x[..., 1::2]
    r1 = x1 * cos - x2 * sin
    r2 = x1 * sin + x2 * cos
    return jnp.concatenate([r1[..., None], r2[..., None]], axis=-1).reshape(x.shape)


def _mm_body(x_ref, w_ref, o_ref):
    o_ref[...] = jnp.dot(x_ref[...], w_ref[...], preferred_element_type=jnp.float32)


def _matmul(x, w, bn):
    M, K = x.shape
    _, N = w.shape
    return pl.pallas_call(
        _mm_body,
        grid=(N // bn,),
        in_specs=[
            pl.BlockSpec((M, K), lambda j: (0, 0)),
            pl.BlockSpec((K, bn), lambda j: (0, j)),
        ],
        out_specs=pl.BlockSpec((M, bn), lambda j: (0, j)),
        out_shape=jax.ShapeDtypeStruct((M, N), jnp.float32),
    )(x, w)


def _attn_body(q_ref, k_ref, v_ref, ck_ref, cv_ref, g_ref, o_ref):
    i = pl.program_id(1)
    qb = q_ref[0]
    kb = k_ref[0]
    vb = v_ref[0]
    scale = DH ** -0.5

    rows = jax.lax.broadcasted_iota(jnp.int32, (QT, T), 0) + i * QT
    cols = jax.lax.broadcasted_iota(jnp.int32, (QT, T), 1)
    causal = rows >= cols
    win = causal & ((rows - cols) < WIN)

    s = jax.lax.dot_general(
        qb, kb, (((1,), (1,)), ((), ())), preferred_element_type=jnp.float32
    ) * scale

    neg = jnp.float32(-1e9)

    def _softmax(m, sc):
        sm = jnp.where(m, sc, neg)
        mx = jnp.max(sm, axis=-1, keepdims=True)
        e = jnp.exp(sm - mx)
        return e / jnp.sum(e, axis=-1, keepdims=True)

    p_s = _softmax(causal, s)
    p_w = _softmax(win, s)
    out_s = jnp.dot(p_s, vb, preferred_element_type=jnp.float32)
    out_w = jnp.dot(p_w, vb, preferred_element_type=jnp.float32)

    ckb = ck_ref[0]
    cvb = cv_ref[0]
    ccols = jax.lax.broadcasted_iota(jnp.int32, (QT, C_PAD), 1)
    crows = jax.lax.broadcasted_iota(jnp.int32, (QT, C_PAD), 0) + i * QT
    c_end = ccols * STRIDE + KERNEL_W - 1
    cmask = (crows >= c_end) & (ccols < NUM_C)
    s_c = jax.lax.dot_general(
        qb, ckb, (((1,), (1,)), ((), ())), preferred_element_type=jnp.float32
    ) * scale
    p_c = _softmax(cmask, s_c)
    valid = (crows[:, :1] >= (KERNEL_W - 1)).astype(jnp.float32)
    out_c = jnp.dot(p_c, cvb, preferred_element_type=jnp.float32) * valid

    g0 = g_ref[0, 0, :][:, None]
    g1 = g_ref[0, 1, :][:, None]
    g2 = g_ref[0, 2, :][:, None]
    o_ref[0] = g0 * out_c + g1 * out_s + g2 * out_w


def _attention(q, k, v, ck, cv, g):
    return pl.pallas_call(
        _attn_body,
        grid=(NQ, T // QT),
        in_specs=[
            pl.BlockSpec((1, QT, DH), lambda h, i: (h, i, 0)),
            pl.BlockSpec((1, T, DH), lambda h, i: (h // G, 0, 0)),
            pl.BlockSpec((1, T, DH), lambda h, i: (h // G, 0, 0)),
            pl.BlockSpec((1, C_PAD, DH), lambda h, i: (h // G, 0, 0)),
            pl.BlockSpec((1, C_PAD, DH), lambda h, i: (h // G, 0, 0)),
            pl.BlockSpec((1, 8, QT), lambda h, i: (h, 0, i)),
        ],
        out_specs=pl.BlockSpec((1, QT, DH), lambda h, i: (h, i, 0)),
        out_shape=jax.ShapeDtypeStruct((NQ, T, DH), jnp.float32),
    )(q, k, v, ck, cv, g)


def kernel(hidden_states, Wq, Wk, Wv, Wo, Wg, w_ck, w_cv):
    B, S, H = hidden_states.shape
    x = hidden_states.reshape(B * S, H)

    Wg_pad = jnp.pad(Wg, ((0, 0), (0, 128 - NQ * 3)))
    W_all = jnp.concatenate([Wq, Wk, Wv, Wg_pad], axis=1)
    qkvg = _matmul(x, W_all, bn=256)
    q = qkvg[:, : NQ * DH].reshape(T, NQ, DH)
    k = qkvg[:, NQ * DH : NQ * DH + NKV * DH].reshape(T, NKV, DH)
    v = qkvg[:, NQ * DH + NKV * DH : NQ * DH + 2 * NKV * DH].reshape(T, NKV, DH)
    g = jax.nn.sigmoid(qkvg[:, 2 * NKV * DH + NQ * DH : 2 * NKV * DH + NQ * DH + NQ * 3])
    g = g.reshape(T, NQ, 3)

    pos = jnp.arange(T)
    q = _rope(q, pos)
    k = _rope(k, pos)

    wk = jax.nn.softmax(w_ck)
    wv = jax.nn.softmax(w_cv)
    k_sub = k.reshape(T // STRIDE, STRIDE, NKV, DH)
    v_sub = v.reshape(T // STRIDE, STRIDE, NKV, DH)
    ck_a = jnp.einsum("cjnd,j->cnd", k_sub, wk[:STRIDE])
    ck_b = jnp.einsum("cjnd,j->cnd", k_sub, wk[STRIDE:])
    cv_a = jnp.einsum("cjnd,j->cnd", v_sub, wv[:STRIDE])
    cv_b = jnp.einsum("cjnd,j->cnd", v_sub, wv[STRIDE:])
    ck = ck_a[:NUM_C] + ck_b[1 : NUM_C + 1]
    cv = cv_a[:NUM_C] + cv_b[1 : NUM_C + 1]
    ck = jnp.pad(ck, ((0, C_PAD - NUM_C), (0, 0), (0, 0))).transpose(1, 0, 2)
    cv = jnp.pad(cv, ((0, C_PAD - NUM_C), (0, 0), (0, 0))).transpose(1, 0, 2)

    qh = q.transpose(1, 0, 2)
    kh = k.transpose(1, 0, 2)
    vh = v.transpose(1, 0, 2)
    g_pad = jnp.pad(g.transpose(1, 2, 0), ((0, 0), (0, 8 - 3), (0, 0)))

    out = _attention(qh, kh, vh, ck, cv, g_pad)
    out = out.transpose(1, 0, 2).reshape(T, NQ * DH)
    y = _matmul(out, Wo, bn=256)
    return y.reshape(B, S, H)


# speedup vs baseline: 1.9701x; 1.9701x over previous
import jax
import jax.numpy as jnp
from jax.experimental import pallas as pl

HIDDEN = 2048
NQ = 16
NKV = 4
DH = 128
G = NQ // NKV
KERNEL_W = 32
STRIDE = 16
WIN = 512
THETA = 500000.0
T = 2048
NUM_C = (T - KERNEL_W) // STRIDE + 1
C_PAD = 128
QT = 256


def _llama3_inv_freq():
    inv = 1.0 / (THETA ** (jnp.arange(0, DH, 2, dtype=jnp.float32) / DH))
    factor, lo, hi, orig = 8.0, 1.0, 4.0, 8192.0
    wavelen = 2.0 * jnp.pi / inv
    smooth = jnp.clip((orig / wavelen - lo) / (hi - lo), 0.0, 1.0)
    return jnp.where(
        wavelen > orig / lo,
        inv / factor,
        jnp.where(wavelen < orig / hi, inv, (1.0 - smooth) * inv / factor + smooth * inv),
    )


def _rope(x, pos):
    inv = _llama3_inv_freq()
    f = pos[:, None].astype(jnp.float32) * inv[None, :]
    cos = jnp.cos(f)[:, None, :]
    sin = jnp.sin(f)[:, None, :]
    x1 = x[..., ::2]
    x2 = x[..., 1::2]
    r1 = x1 * cos - x2 * sin
    r2 = x1 * sin + x2 * cos
    return jnp.concatenate([r1[..., None], r2[..., None]], axis=-1).reshape(x.shape)


def _mm_body(x_ref, w_ref, o_ref):
    o_ref[...] = jnp.dot(x_ref[...], w_ref[...], preferred_element_type=jnp.float32)


def _matmul(x, w, bn):
    M, K = x.shape
    _, N = w.shape
    return pl.pallas_call(
        _mm_body,
        grid=(N // bn,),
        in_specs=[
            pl.BlockSpec((M, K), lambda j: (0, 0)),
            pl.BlockSpec((K, bn), lambda j: (0, j)),
        ],
        out_specs=pl.BlockSpec((M, bn), lambda j: (0, j)),
        out_shape=jax.ShapeDtypeStruct((M, N), jnp.float32),
    )(x, w)


def _attn_body(q_ref, k_ref, v_ref, ck_ref, cv_ref, g_ref, o_ref):
    i = pl.program_id(1)
    qb = q_ref[0]
    kb = k_ref[0]
    vb = v_ref[0]
    scale = DH ** -0.5

    rows = jax.lax.broadcasted_iota(jnp.int32, (QT, T), 0) + i * QT
    cols = jax.lax.broadcasted_iota(jnp.int32, (QT, T), 1)
    causal = rows >= cols
    win = causal & ((rows - cols) < WIN)

    s = jax.lax.dot_general(
        qb, kb, (((1,), (1,)), ((), ())), preferred_element_type=jnp.float32
    ) * scale

    neg = jnp.float32(-1e9)

    def _softmax(m, sc):
        sm = jnp.where(m, sc, neg)
        mx = jnp.max(sm, axis=-1, keepdims=True)
        e = jnp.exp(sm - mx)
        return e / jnp.sum(e, axis=-1, keepdims=True)

    p_s = _softmax(causal, s)
    p_w = _softmax(win, s)
    out_s = jnp.dot(p_s, vb, preferred_element_type=jnp.float32)
    out_w = jnp.dot(p_w, vb, preferred_element_type=jnp.float32)

    ckb = ck_ref[0]
    cvb = cv_ref[0]
    ccols = jax.lax.broadcasted_iota(jnp.int32, (QT, C_PAD), 1)
    crows = jax.lax.broadcasted_iota(jnp.int32, (QT, C_PAD), 0) + i * QT
    c_end = ccols * STRIDE + KERNEL_W - 1
    cmask = (crows >= c_end) & (ccols < NUM_C)
    s_c = jax.lax.dot_general(
        qb, ckb, (((1,), (1,)), ((), ())), preferred_element_type=jnp.float32
    ) * scale
    p_c = _softmax(cmask, s_c)
    valid = (crows[:, :1] >= (KERNEL_W - 1)).astype(jnp.float32)
    out_c = jnp.dot(p_c, cvb, preferred_element_type=jnp.float32) * valid

    g0 = g_ref[0, 0, :][:, None]
    g1 = g_ref[0, 1, :][:, None]
    g2 = g_ref[0, 2, :][:, None]
    o_ref[0] = g0 * out_c + g1 * out_s + g2 * out_w


def _attention(q, k, v, ck, cv, g):
    return pl.pallas_call(
        _attn_body,
        grid=(NQ, T // QT),
        in_specs=[
            pl.BlockSpec((1, QT, DH), lambda h, i: (h, i, 0)),
            pl.BlockSpec((1, T, DH), lambda h, i: (h // G, 0, 0)),
            pl.BlockSpec((1, T, DH), lambda h, i: (h // G, 0, 0)),
            pl.BlockSpec((1, C_PAD, DH), lambda h, i: (h // G, 0, 0)),
            pl.BlockSpec((1, C_PAD, DH), lambda h, i: (h // G, 0, 0)),
            pl.BlockSpec((1, 8, QT), lambda h, i: (h, 0, i)),
        ],
        out_specs=pl.BlockSpec((1, QT, DH), lambda h, i: (h, i, 0)),
        out_shape=jax.ShapeDtypeStruct((NQ, T, DH), jnp.float32),
    )(q, k, v, ck, cv, g)


def kernel(hidden_states, Wq, Wk, Wv, Wo, Wg, w_ck, w_cv):
    B, S, H = hidden_states.shape
    x = hidden_states.reshape(B * S, H)

    Wg_pad = jnp.pad(Wg, ((0, 0), (0, 128 - NQ * 3)))
    W_all = jnp.concatenate([Wq, Wk, Wv, Wg_pad], axis=1)
    qkvg = _matmul(x, W_all, bn=640)
    q = qkvg[:, : NQ * DH].reshape(T, NQ, DH)
    k = qkvg[:, NQ * DH : NQ * DH + NKV * DH].reshape(T, NKV, DH)
    v = qkvg[:, NQ * DH + NKV * DH : NQ * DH + 2 * NKV * DH].reshape(T, NKV, DH)
    g = jax.nn.sigmoid(qkvg[:, 2 * NKV * DH + NQ * DH : 2 * NKV * DH + NQ * DH + NQ * 3])
    g = g.reshape(T, NQ, 3)

    pos = jnp.arange(T)
    q = _rope(q, pos)
    k = _rope(k, pos)

    wk = jax.nn.softmax(w_ck)
    wv = jax.nn.softmax(w_cv)
    k_sub = k.reshape(T // STRIDE, STRIDE, NKV, DH)
    v_sub = v.reshape(T // STRIDE, STRIDE, NKV, DH)
    ck_a = jnp.einsum("cjnd,j->cnd", k_sub, wk[:STRIDE])
    ck_b = jnp.einsum("cjnd,j->cnd", k_sub, wk[STRIDE:])
    cv_a = jnp.einsum("cjnd,j->cnd", v_sub, wv[:STRIDE])
    cv_b = jnp.einsum("cjnd,j->cnd", v_sub, wv[STRIDE:])
    ck = ck_a[:NUM_C] + ck_b[1 : NUM_C + 1]
    cv = cv_a[:NUM_C] + cv_b[1 : NUM_C + 1]
    ck = jnp.pad(ck, ((0, C_PAD - NUM_C), (0, 0), (0, 0))).transpose(1, 0, 2)
    cv = jnp.pad(cv, ((0, C_PAD - NUM_C), (0, 0), (0, 0))).transpose(1, 0, 2)

    qh = q.transpose(1, 0, 2)
    kh = k.transpose(1, 0, 2)
    vh = v.transpose(1, 0, 2)
    g_pad = jnp.pad(g.transpose(1, 2, 0), ((0, 0), (0, 8 - 3), (0, 0)))

    out = _attention(qh, kh, vh, ck, cv, g_pad)
    out = out.transpose(1, 0, 2).reshape(T, NQ * DH)
    y = _matmul(out, Wo, bn=256)
    return y.reshape(B, S, H)


# speedup vs baseline: 2.0013x; 1.0158x over previous
import jax
import jax.numpy as jnp
from jax.experimental import pallas as pl

HIDDEN = 2048
NQ = 16
NKV = 4
DH = 128
G = NQ // NKV
KERNEL_W = 32
STRIDE = 16
WIN = 512
THETA = 500000.0
T = 2048
NUM_C = (T - KERNEL_W) // STRIDE + 1
C_PAD = 128
QT = 256


def _llama3_inv_freq():
    inv = 1.0 / (THETA ** (jnp.arange(0, DH, 2, dtype=jnp.float32) / DH))
    factor, lo, hi, orig = 8.0, 1.0, 4.0, 8192.0
    wavelen = 2.0 * jnp.pi / inv
    smooth = jnp.clip((orig / wavelen - lo) / (hi - lo), 0.0, 1.0)
    return jnp.where(
        wavelen > orig / lo,
        inv / factor,
        jnp.where(wavelen < orig / hi, inv, (1.0 - smooth) * inv / factor + smooth * inv),
    )


def _rope(x, pos):
    inv = _llama3_inv_freq()
    f = pos[:, None].astype(jnp.float32) * inv[None, :]
    cos = jnp.cos(f)[:, None, :]
    sin = jnp.sin(f)[:, None, :]
    x1 = x[..., ::2]
    x2 = x[..., 1::2]
    r1 = x1 * cos - x2 * sin
    r2 = x1 * sin + x2 * cos
    return jnp.concatenate([r1[..., None], r2[..., None]], axis=-1).reshape(x.shape)


def _mm_body(x_ref, w_ref, o_ref):
    o_ref[...] = jnp.dot(x_ref[...], w_ref[...], preferred_element_type=jnp.float32)


def _matmul(x, w, bn):
    M, K = x.shape
    _, N = w.shape
    return pl.pallas_call(
        _mm_body,
        grid=(N // bn,),
        in_specs=[
            pl.BlockSpec((M, K), lambda j: (0, 0)),
            pl.BlockSpec((K, bn), lambda j: (0, j)),
        ],
        out_specs=pl.BlockSpec((M, bn), lambda j: (0, j)),
        out_shape=jax.ShapeDtypeStruct((M, N), jnp.float32),
    )(x, w)


def _attn_body(q_ref, k_ref, v_ref, ck_ref, cv_ref, g_ref, o_ref):
    i = pl.program_id(1)
    qb = q_ref[0]
    kb = k_ref[0]
    vb = v_ref[0]
    scale = DH ** -0.5

    rows = jax.lax.broadcasted_iota(jnp.int32, (QT, T), 0) + i * QT
    cols = jax.lax.broadcasted_iota(jnp.int32, (QT, T), 1)
    causal = rows >= cols
    win = causal & ((rows - cols) < WIN)

    s = jax.lax.dot_general(
        qb, kb, (((1,), (1,)), ((), ())), preferred_element_type=jnp.float32
    ) * scale

    neg = jnp.float32(-1e9)

    def _softmax(m, sc):
        sm = jnp.where(m, sc, neg)
        mx = jnp.max(sm, axis=-1, keepdims=True)
        e = jnp.exp(sm - mx)
        return e / jnp.sum(e, axis=-1, keepdims=True)

    p_s = _softmax(causal, s).astype(jnp.bfloat16)
    p_w = _softmax(win, s).astype(jnp.bfloat16)
    out_s = jnp.dot(p_s, vb, preferred_element_type=jnp.float32)
    out_w = jnp.dot(p_w, vb, preferred_element_type=jnp.float32)

    ckb = ck_ref[0]
    cvb = cv_ref[0]
    ccols = jax.lax.broadcasted_iota(jnp.int32, (QT, C_PAD), 1)
    crows = jax.lax.broadcasted_iota(jnp.int32, (QT, C_PAD), 0) + i * QT
    c_end = ccols * STRIDE + KERNEL_W - 1
    cmask = (crows >= c_end) & (ccols < NUM_C)
    s_c = jax.lax.dot_general(
        qb, ckb, (((1,), (1,)), ((), ())), preferred_element_type=jnp.float32
    ) * scale
    p_c = _softmax(cmask, s_c).astype(jnp.bfloat16)
    valid = (crows[:, :1] >= (KERNEL_W - 1)).astype(jnp.float32)
    out_c = jnp.dot(p_c, cvb, preferred_element_type=jnp.float32) * valid

    g0 = g_ref[0, 0, :][:, None]
    g1 = g_ref[0, 1, :][:, None]
    g2 = g_ref[0, 2, :][:, None]
    o_ref[0] = g0 * out_c + g1 * out_s + g2 * out_w


def _attention(q, k, v, ck, cv, g):
    return pl.pallas_call(
        _attn_body,
        grid=(NQ, T // QT),
        in_specs=[
            pl.BlockSpec((1, QT, DH), lambda h, i: (h, i, 0)),
            pl.BlockSpec((1, T, DH), lambda h, i: (h // G, 0, 0)),
            pl.BlockSpec((1, T, DH), lambda h, i: (h // G, 0, 0)),
            pl.BlockSpec((1, C_PAD, DH), lambda h, i: (h // G, 0, 0)),
            pl.BlockSpec((1, C_PAD, DH), lambda h, i: (h // G, 0, 0)),
            pl.BlockSpec((1, 8, QT), lambda h, i: (h, 0, i)),
        ],
        out_specs=pl.BlockSpec((1, QT, DH), lambda h, i: (h, i, 0)),
        out_shape=jax.ShapeDtypeStruct((NQ, T, DH), jnp.float32),
    )(q, k, v, ck, cv, g)


def kernel(hidden_states, Wq, Wk, Wv, Wo, Wg, w_ck, w_cv):
    B, S, H = hidden_states.shape
    x = hidden_states.reshape(B * S, H)

    Wg_pad = jnp.pad(Wg, ((0, 0), (0, 128 - NQ * 3)))
    W_all = jnp.concatenate([Wq, Wk, Wv, Wg_pad], axis=1)
    qkvg = _matmul(x.astype(jnp.bfloat16), W_all.astype(jnp.bfloat16), bn=640)
    q = qkvg[:, : NQ * DH].reshape(T, NQ, DH)
    k = qkvg[:, NQ * DH : NQ * DH + NKV * DH].reshape(T, NKV, DH)
    v = qkvg[:, NQ * DH + NKV * DH : NQ * DH + 2 * NKV * DH].reshape(T, NKV, DH)
    g = jax.nn.sigmoid(qkvg[:, 2 * NKV * DH + NQ * DH : 2 * NKV * DH + NQ * DH + NQ * 3])
    g = g.reshape(T, NQ, 3)

    pos = jnp.arange(T)
    q = _rope(q, pos)
    k = _rope(k, pos)

    wk = jax.nn.softmax(w_ck)
    wv = jax.nn.softmax(w_cv)
    k_sub = k.reshape(T // STRIDE, STRIDE, NKV, DH)
    v_sub = v.reshape(T // STRIDE, STRIDE, NKV, DH)
    ck_a = jnp.einsum("cjnd,j->cnd", k_sub, wk[:STRIDE])
    ck_b = jnp.einsum("cjnd,j->cnd", k_sub, wk[STRIDE:])
    cv_a = jnp.einsum("cjnd,j->cnd", v_sub, wv[:STRIDE])
    cv_b = jnp.einsum("cjnd,j->cnd", v_sub, wv[STRIDE:])
    ck = ck_a[:NUM_C] + ck_b[1 : NUM_C + 1]
    cv = cv_a[:NUM_C] + cv_b[1 : NUM_C + 1]
    ck = jnp.pad(ck, ((0, C_PAD - NUM_C), (0, 0), (0, 0))).transpose(1, 0, 2)
    cv = jnp.pad(cv, ((0, C_PAD - NUM_C), (0, 0), (0, 0))).transpose(1, 0, 2)

    qh = q.transpose(1, 0, 2).astype(jnp.bfloat16)
    kh = k.transpose(1, 0, 2).astype(jnp.bfloat16)
    vh = v.transpose(1, 0, 2).astype(jnp.bfloat16)
    ck = ck.astype(jnp.bfloat16)
    cv = cv.astype(jnp.bfloat16)
    g_pad = jnp.pad(g.transpose(1, 2, 0), ((0, 0), (0, 8 - 3), (0, 0)))

    out = _attention(qh, kh, vh, ck, cv, g_pad)
    out = out.transpose(1, 0, 2).reshape(T, NQ * DH)
    y = _matmul(out.astype(jnp.bfloat16), Wo.astype(jnp.bfloat16), bn=256)
    return y.reshape(B, S, H)


# speedup vs baseline: 3.3210x; 1.6594x over previous
import jax
import jax.numpy as jnp
from jax.experimental import pallas as pl
from jax.experimental.pallas import tpu as pltpu

HIDDEN = 2048
NQ = 16
NKV = 4
DH = 128
G = NQ // NKV
KERNEL_W = 32
STRIDE = 16
WIN = 512
THETA = 500000.0
T = 2048
NUM_C = (T - KERNEL_W) // STRIDE + 1
C_PAD = 128
QT = 256
WCOLS = 3 * QT


def _llama3_inv_freq():
    inv = 1.0 / (THETA ** (jnp.arange(0, DH, 2, dtype=jnp.float32) / DH))
    factor, lo, hi, orig = 8.0, 1.0, 4.0, 8192.0
    wavelen = 2.0 * jnp.pi / inv
    smooth = jnp.clip((orig / wavelen - lo) / (hi - lo), 0.0, 1.0)
    return jnp.where(
        wavelen > orig / lo,
        inv / factor,
        jnp.where(wavelen < orig / hi, inv, (1.0 - smooth) * inv / factor + smooth * inv),
    )


def _prep_body(x_ref, wq_ref, wk_ref, wv_ref, wg_ref, cos_ref, sin_ref,
               q_ref, k_ref, v_ref, g_ref):
    xb = x_ref[...].astype(jnp.bfloat16)
    qp = jnp.dot(xb, wq_ref[...], preferred_element_type=jnp.float32)
    kp = jnp.dot(xb, wk_ref[...], preferred_element_type=jnp.float32)
    vp = jnp.dot(xb, wv_ref[...], preferred_element_type=jnp.float32)
    gp = jnp.dot(xb, wg_ref[...], preferred_element_type=jnp.float32)
    g_ref[...] = jax.nn.sigmoid(gp)
    cos = cos_ref[...]
    sin = sin_ref[...]
    scale = DH ** -0.5
    for h in range(NQ):
        x1 = qp[:, h * DH : h * DH + 64]
        x2 = qp[:, h * DH + 64 : (h + 1) * DH]
        r = jnp.concatenate([x1 * cos - x2 * sin, x1 * sin + x2 * cos], axis=1)
        q_ref[h] = (r * scale).astype(jnp.bfloat16)
    for n in range(NKV):
        x1 = kp[:, n * DH : n * DH + 64]
        x2 = kp[:, n * DH + 64 : (n + 1) * DH]
        r = jnp.concatenate([x1 * cos - x2 * sin, x1 * sin + x2 * cos], axis=1)
        k_ref[n] = r.astype(jnp.bfloat16)
        v_ref[n] = vp[:, n * DH : (n + 1) * DH].astype(jnp.bfloat16)


def _prep(x, wq_p, wk_p, wv, wg_pad, cos, sin):
    return pl.pallas_call(
        _prep_body,
        grid=(T // QT,),
        in_specs=[
            pl.BlockSpec((QT, HIDDEN), lambda i: (i, 0)),
            pl.BlockSpec((HIDDEN, NQ * DH), lambda i: (0, 0)),
            pl.BlockSpec((HIDDEN, NKV * DH), lambda i: (0, 0)),
            pl.BlockSpec((HIDDEN, NKV * DH), lambda i: (0, 0)),
            pl.BlockSpec((HIDDEN, 128), lambda i: (0, 0)),
            pl.BlockSpec((QT, 64), lambda i: (i, 0)),
            pl.BlockSpec((QT, 64), lambda i: (i, 0)),
        ],
        out_specs=[
            pl.BlockSpec((NQ, QT, DH), lambda i: (0, i, 0)),
            pl.BlockSpec((NKV, QT, DH), lambda i: (0, i, 0)),
            pl.BlockSpec((NKV, QT, DH), lambda i: (0, i, 0)),
            pl.BlockSpec((QT, 128), lambda i: (i, 0)),
        ],
        out_shape=[
            jax.ShapeDtypeStruct((NQ, T, DH), jnp.bfloat16),
            jax.ShapeDtypeStruct((NKV, T, DH), jnp.bfloat16),
            jax.ShapeDtypeStruct((NKV, T, DH), jnp.bfloat16),
            jax.ShapeDtypeStruct((T, 128), jnp.float32),
        ],
    )(x, wq_p, wk_p, wv, wg_pad, cos, sin)


def _attn_body(q_ref, k_ref, v_ref, ck_ref, cv_ref, g_ref, o_ref, e_ref):
    i = pl.program_id(1)
    qb = q_ref[0]
    kb = k_ref[0]

    rows = jax.lax.broadcasted_iota(jnp.int32, (QT, T), 0) + i * QT
    cols = jax.lax.broadcasted_iota(jnp.int32, (QT, T), 1)
    causal = rows >= cols

    s = jax.lax.dot_general(
        qb, kb, (((1,), (1,)), ((), ())), preferred_element_type=jnp.float32
    )
    s = jnp.where(causal, s, jnp.float32(-1e9))
    mx = jnp.max(s, axis=-1, keepdims=True)
    e = jnp.exp(s - mx)
    l_s = jnp.sum(e, axis=-1, keepdims=True)
    e16 = e.astype(jnp.bfloat16)
    out_s = jnp.dot(e16, v_ref[0], preferred_element_type=jnp.float32) / l_s

    wstart = jnp.maximum(i - 2, 0) * QT
    e_ref[...] = e
    ew = e_ref[:, pl.ds(wstart, WCOLS)]
    wcols = jax.lax.broadcasted_iota(jnp.int32, (QT, WCOLS), 1)
    wrows = jax.lax.broadcasted_iota(jnp.int32, (QT, WCOLS), 0) + (i * QT - wstart)
    ew = jnp.where((wrows - wcols) < WIN, ew, jnp.float32(0.0))
    l_w = jnp.sum(ew, axis=-1, keepdims=True)
    vw = v_ref[0, pl.ds(wstart, WCOLS), :]
    out_w = jnp.dot(ew.astype(jnp.bfloat16), vw, preferred_element_type=jnp.float32) / l_w

    ccols = jax.lax.broadcasted_iota(jnp.int32, (QT, C_PAD), 1)
    crows = jax.lax.broadcasted_iota(jnp.int32, (QT, C_PAD), 0) + i * QT
    cmask = (crows >= ccols * STRIDE + KERNEL_W - 1) & (ccols < NUM_C)
    s_c = jax.lax.dot_general(
        qb, ck_ref[0], (((1,), (1,)), ((), ())), preferred_element_type=jnp.float32
    )
    s_c = jnp.where(cmask, s_c, jnp.float32(-1e9))
    mc = jnp.max(s_c, axis=-1, keepdims=True)
    ec = jnp.exp(s_c - mc)
    l_c = jnp.sum(ec, axis=-1, keepdims=True)
    valid = (crows[:, :1] >= (KERNEL_W - 1)).astype(jnp.float32)
    out_c = jnp.dot(ec.astype(jnp.bfloat16), cv_ref[0], preferred_element_type=jnp.float32)
    out_c = out_c * (valid / l_c)

    g0 = g_ref[0, 0, :][:, None]
    g1 = g_ref[0, 1, :][:, None]
    g2 = g_ref[0, 2, :][:, None]
    o_ref[...] = (g0 * out_c + g1 * out_s + g2 * out_w).astype(jnp.bfloat16)


def _attention(q, k, v, ck, cv, g):
    return pl.pallas_call(
        _attn_body,
        grid=(NQ, T // QT),
        in_specs=[
            pl.BlockSpec((1, QT, DH), lambda h, i: (h, i, 0)),
            pl.BlockSpec((1, T, DH), lambda h, i: (h // G, 0, 0)),
            pl.BlockSpec((1, T, DH), lambda h, i: (h // G, 0, 0)),
            pl.BlockSpec((1, C_PAD, DH), lambda h, i: (h // G, 0, 0)),
            pl.BlockSpec((1, C_PAD, DH), lambda h, i: (h // G, 0, 0)),
            pl.BlockSpec((1, 8, QT), lambda h, i: (h, 0, i)),
        ],
        out_specs=pl.BlockSpec((QT, DH), lambda h, i: (i, h)),
        out_shape=jax.ShapeDtypeStruct((T, NQ * DH), jnp.bfloat16),
        scratch_shapes=[pltpu.VMEM((QT, T), jnp.float32)],
    )(q, k, v, ck, cv, g)


def _mm_body(x_ref, w_ref, o_ref):
    o_ref[...] = jnp.dot(x_ref[...], w_ref[...], preferred_element_type=jnp.float32)


def _matmul(x, w, bn):
    M, K = x.shape
    _, N = w.shape
    return pl.pallas_call(
        _mm_body,
        grid=(N // bn,),
        in_specs=[
            pl.BlockSpec((M, K), lambda j: (0, 0)),
            pl.BlockSpec((K, bn), lambda j: (0, j)),
        ],
        out_specs=pl.BlockSpec((M, bn), lambda j: (0, j)),
        out_shape=jax.ShapeDtypeStruct((M, N), jnp.float32),
    )(x, w)


def _deinterleave_cols(w, nheads):
    return w.reshape(HIDDEN, nheads, 64, 2).transpose(0, 1, 3, 2).reshape(HIDDEN, nheads * DH)


def kernel(hidden_states, Wq, Wk, Wv, Wo, Wg, w_ck, w_cv):
    B, S, H = hidden_states.shape
    x = hidden_states.reshape(B * S, H)

    wq_p = _deinterleave_cols(Wq, NQ).astype(jnp.bfloat16)
    wk_p = _deinterleave_cols(Wk, NKV).astype(jnp.bfloat16)
    wv_b = Wv.astype(jnp.bfloat16)
    wg_pad = jnp.pad(Wg, ((0, 0), (0, 128 - NQ * 3))).astype(jnp.bfloat16)

    pos = jnp.arange(T, dtype=jnp.float32)
    f = pos[:, None] * _llama3_inv_freq()[None, :]
    cos = jnp.cos(f)
    sin = jnp.sin(f)

    qh, kh, vh, gsig = _prep(x, wq_p, wk_p, wv_b, wg_pad, cos, sin)

    wk_c = jax.nn.softmax(w_ck)
    wv_c = jax.nn.softmax(w_cv)
    k_sub = kh.reshape(NKV, T // STRIDE, STRIDE, DH).astype(jnp.float32)
    v_sub = vh.reshape(NKV, T // STRIDE, STRIDE, DH).astype(jnp.float32)
    ck_a = jnp.einsum("ncjd,j->ncd", k_sub, wk_c[:STRIDE])
    ck_b = jnp.einsum("ncjd,j->ncd", k_sub, wk_c[STRIDE:])
    cv_a = jnp.einsum("ncjd,j->ncd", v_sub, wv_c[:STRIDE])
    cv_b = jnp.einsum("ncjd,j->ncd", v_sub, wv_c[STRIDE:])
    ck = ck_a[:, :NUM_C] + ck_b[:, 1 : NUM_C + 1]
    cv = cv_a[:, :NUM_C] + cv_b[:, 1 : NUM_C + 1]
    ck = jnp.pad(ck, ((0, 0), (0, C_PAD - NUM_C), (0, 0))).astype(jnp.bfloat16)
    cv = jnp.pad(cv, ((0, 0), (0, C_PAD - NUM_C), (0, 0))).astype(jnp.bfloat16)

    g_pad = jnp.pad(
        gsig[:, : NQ * 3].reshape(T, NQ, 3).transpose(1, 2, 0), ((0, 0), (0, 5), (0, 0))
    )

    out = _attention(qh, kh, vh, ck, cv, g_pad)
    y = _matmul(out, Wo.astype(jnp.bfloat16), bn=256)
    return y.reshape(B, S, H)
